# baseline (speedup 1.0000x reference)
import functools

import jax
import jax.numpy as jnp
from jax import lax
from jax.experimental import pallas as pl
from jax.experimental.pallas import tpu as pltpu
from jax.experimental.pallas import tpu_sc as plsc

N_ROWS = 16384
N_MEM = 100000
D = 128
_L = 16
_NW = 32
_B_W = N_ROWS // _NW
_CH = 128
_NCH = _B_W // _CH
_P = 8
_KS = N_ROWS // _P
_TBL = N_MEM + _L
_VHALF = N_MEM // 2


def _norm_body(val_ref, out_ref):
    v = val_ref[...] * (1.0 / 255.0)
    mean = jnp.mean(v, axis=0, keepdims=True)
    mn = jnp.min(v, axis=0, keepdims=True)
    mx = jnp.max(v, axis=0, keepdims=True)
    out_ref[...] = (v - mean) / jnp.abs(mx - mn)


def _normalize(val):
    return pl.pallas_call(
        _norm_body,
        out_shape=jax.ShapeDtypeStruct((N_ROWS, D), jnp.float32),
    )(val)


_mesh = plsc.VectorSubcoreMesh(core_axis_name="c", subcore_axis_name="s")


@functools.partial(
    pl.kernel,
    mesh=_mesh,
    out_type=jax.ShapeDtypeStruct((_P * _TBL,), jnp.int32),
    compiler_params=pltpu.CompilerParams(needs_layout_passes=False),
    scratch_types=[
        pltpu.VMEM((_KS,), jnp.int32),
        pltpu.VMEM((_KS // 128, 128), jnp.int32),
        pltpu.VMEM((_KS // 128, 128), jnp.int32),
        pltpu.VMEM((N_MEM,), jnp.int32),
        pltpu.SemaphoreType.DMA,
    ],
)
def _sc_slots(idx_hbm, slot_hbm, idx_v, siv_v, kv_v, table_v, cs):
    c = lax.axis_index("c")
    s = lax.axis_index("s")
    lane = lax.iota(jnp.int32, _L)
    p = c * (_P // 2) + s

    @pl.when(s < _P // 2)
    def _():
        kbase = p * _KS
        pltpu.sync_copy(idx_hbm.at[pl.ds(kbase, _KS)], idx_v)
        lane_next = jnp.minimum(lane + 1, _L - 1)

        def scat_body(g, carry):
            for u in range(2):
                st = pl.multiple_of((g * 2 + u) * _L, _L)
                iv = idx_v[pl.ds(st, _L)]
                key = (iv << 14) | (lane + st + kbase)
                skey = lax.sort(key)
                nxt = skey.at[lane_next].get(mode="promise_in_bounds")
                keep = ((skey >> 14) != (nxt >> 14)) | (lane == _L - 1)
                plsc.store_scatter(
                    table_v, [skey >> 14], skey & 0x3FFF, mask=keep)
            return carry

        lax.fori_loop(0, _KS // _L // 2, scat_body, None)

        tbase = p * _TBL
        trash = tbase + N_MEM + lane
        for r in range(_KS // 128):
            def ext_body(g, carry, r=r):
                col = pl.multiple_of(g * _L, _L)
                st = r * 128 + g * _L
                iv = idx_v[pl.ds(pl.multiple_of(st, _L), _L)]
                kvec = lane + st + kbase
                v = plsc.load_gather(table_v, [iv])
                good = v == kvec
                siv_v[r, pl.ds(col, _L)] = jnp.where(good, tbase + iv, trash)
                kv_v[r, pl.ds(col, _L)] = kvec
                return carry

            lax.fori_loop(0, 128 // _L, ext_body, None)

        hs = [
            pltpu.async_copy(kv_v.at[r], slot_hbm.at[siv_v.at[r]], cs)
            for r in range(_KS // 128)
        ]
        for h in hs:
            h.wait()


@functools.partial(
    pl.kernel,
    mesh=_mesh,
    out_type=jax.ShapeDtypeStruct((N_ROWS, D), jnp.float32),
    compiler_params=pltpu.CompilerParams(needs_layout_passes=False),
    scratch_types=[
        pltpu.VMEM((_B_W,), jnp.int32),
        pltpu.VMEM((N_ROWS,), jnp.int32),
        pltpu.VMEM((_P * _B_W,), jnp.int32),
        pltpu.VMEM((_P * _B_W,), jnp.int32),
        pltpu.VMEM((_B_W,), jnp.int32),
        pltpu.VMEM((_NCH, _CH, D), jnp.float32),
        pltpu.SemaphoreType.DMA,
        pltpu.SemaphoreType.DMA,
        pltpu.SemaphoreType.DMA,
        pltpu.SemaphoreType.DMA,
        pltpu.SemaphoreType.DMA,
        pltpu.SemaphoreType.DMA,
        pltpu.SemaphoreType.DMA,
        pltpu.SemaphoreType.DMA,
    ],
)
def _sc_gather(idx_hbm, slot_hbm, vn_hbm, out_hbm, idx_v, idxf_v, ql_v,
               kc_v, kpos_v, rows_v, ps, fs, gs0, gs1, gs2, gs3, ws0, ws1):
    c = lax.axis_index("c")
    s = lax.axis_index("s")
    wid = s * 2 + c
    base = wid * _B_W
    hf = pltpu.async_copy(idx_hbm, idxf_v, fs)
    pltpu.sync_copy(idx_hbm.at[pl.ds(base, _B_W)], idx_v)

    def ql_body(j, carry):
        col = pl.multiple_of(j * _L, _L)
        v = idx_v[pl.ds(col, _L)]
        for p in range(_P):
            dst = pl.multiple_of(p * _B_W + col, _L)
            ql_v[pl.ds(dst, _L)] = v + p * _TBL
        return carry

    lax.fori_loop(0, _B_W // _L, ql_body, None)

    hc = [
        pltpu.async_copy(
            slot_hbm.at[ql_v.at[pl.ds(p * _B_W, _B_W)]],
            kc_v.at[pl.ds(p * _B_W, _B_W)], ps)
        for p in range(_P)
    ]
    for h in hc:
        h.wait()
    hf.wait()

    def mg_body(j, carry):
        col = pl.multiple_of(j * _L, _L)
        v = idx_v[pl.ds(col, _L)]
        m = jnp.zeros((_L,), jnp.int32)
        for p in range(_P):
            kc = kc_v[pl.ds(pl.multiple_of(p * _B_W + col, _L), _L)]
            w = plsc.load_gather(idxf_v, [kc & (N_ROWS - 1)])
            ok = ((kc >> 11) == p) & (w == v)
            m = jnp.where(ok, jnp.maximum(m, kc), m)
        kpos_v[pl.ds(col, _L)] = m
        return carry

    lax.fori_loop(0, _B_W // _L, mg_body, None)

    gsem = (gs0, gs1, gs2, gs3)
    wsem = (ws0, ws1)

    hg = [
        pltpu.async_copy(
            vn_hbm.at[kpos_v.at[pl.ds(j * _CH, _CH)]], rows_v.at[j],
            gsem[j])
        for j in range(_NCH)
    ]
    hw = []
    for j in range(_NCH):
        hg[j].wait()
        hw.append(pltpu.async_copy(
            rows_v.at[j], out_hbm.at[pl.ds(base + j * _CH, _CH)],
            wsem[j & 1]))
    for h in hw:
        h.wait()


def kernel(mem, val, idx):
    del mem
    idx32 = idx.astype(jnp.int32)
    vn = _normalize(val)
    slots = _sc_slots(idx32)
    out = _sc_gather(idx32, slots, vn)
    return out


# speedup vs baseline: 1.2243x; 1.2243x over previous
import functools

import jax
import jax.numpy as jnp
from jax import lax
from jax.experimental import pallas as pl
from jax.experimental.pallas import tpu as pltpu
from jax.experimental.pallas import tpu_sc as plsc

N_ROWS = 16384
N_MEM = 100000
D = 128
_L = 16
_NW = 32
_B_W = N_ROWS // _NW
_CH = 128
_NCH = _B_W // _CH
_PCH = 2048
_P = 8
_KS = N_ROWS // _P


def _norm_body(val_ref, out_ref):
    v = val_ref[...] * (1.0 / 255.0)
    mean = jnp.mean(v, axis=0, keepdims=True)
    mn = jnp.min(v, axis=0, keepdims=True)
    mx = jnp.max(v, axis=0, keepdims=True)
    out_ref[...] = (v - mean) / jnp.abs(mx - mn)


def _normalize(val):
    return pl.pallas_call(
        _norm_body,
        out_shape=jax.ShapeDtypeStruct((N_ROWS, D), jnp.float32),
    )(val)


_mesh = plsc.VectorSubcoreMesh(core_axis_name="c", subcore_axis_name="s")


@functools.partial(
    pl.kernel,
    mesh=_mesh,
    out_type=jax.ShapeDtypeStruct((_P, N_ROWS), jnp.int32),
    compiler_params=pltpu.CompilerParams(needs_layout_passes=False),
    scratch_types=[
        pltpu.VMEM((N_ROWS,), jnp.int32),
        pltpu.VMEM((2, _PCH), jnp.int32),
        pltpu.VMEM((N_MEM,), jnp.int32),
        pltpu.SemaphoreType.DMA,
        pltpu.SemaphoreType.DMA,
    ],
)
def _sc_pos(idx_hbm, pos_hbm, idx_v, pos_b, table_v, ws0, ws1):
    c = lax.axis_index("c")
    s = lax.axis_index("s")
    p = c * (_P // 2) + s

    @pl.when(s < _P // 2)
    def _():
        pltpu.sync_copy(idx_hbm, idx_v)
        lane = lax.iota(jnp.int32, _L)
        kbase = p * _KS
        zero_v = jnp.zeros((_L,), jnp.int32)

        def init_body(j, carry):
            for u in range(8):
                st = pl.multiple_of((j * 8 + u) * _L, _L)
                plsc.store_scatter(table_v, [idx_v[pl.ds(st, _L)]], zero_v)
            return carry

        lax.fori_loop(0, N_ROWS // _L // 8, init_body, None)

        lane_next = jnp.minimum(lane + 1, _L - 1)

        def scat_body(g, carry):
            for u in range(2):
                st = pl.multiple_of(kbase + (g * 2 + u) * _L, _L)
                iv = idx_v[pl.ds(st, _L)]
                key = (iv << 14) | (lane + st)
                skey = lax.sort(key)
                nxt = skey.at[lane_next].get(mode="promise_in_bounds")
                keep = ((skey >> 14) != (nxt >> 14)) | (lane == _L - 1)
                plsc.store_scatter(
                    table_v, [skey >> 14], skey & 0x3FFF, mask=keep)
            return carry

        lax.fori_loop(0, _KS // _L // 2, scat_body, None)

        wsem = (ws0, ws1)
        wh = [None] * (N_ROWS // _PCH)
        for ci in range(N_ROWS // _PCH):
            b = ci & 1
            if ci >= 2:
                wh[ci - 2].wait()

            def lk_body(j, inner, ci=ci, b=b):
                for u in range(8):
                    st2 = pl.multiple_of(
                        ci * _PCH + (j * 8 + u) * _L, _L)
                    iv = idx_v[pl.ds(st2, _L)]
                    dst = pl.multiple_of((j * 8 + u) * _L, _L)
                    pos_b[b, pl.ds(dst, _L)] = plsc.load_gather(
                        table_v, [iv])
                return inner

            lax.fori_loop(0, _PCH // _L // 8, lk_body, None)
            wh[ci] = pltpu.async_copy(
                pos_b.at[b], pos_hbm.at[p, pl.ds(ci * _PCH, _PCH)],
                wsem[b])
        wh[-2].wait()
        wh[-1].wait()


@functools.partial(
    pl.kernel,
    mesh=_mesh,
    out_type=jax.ShapeDtypeStruct((N_ROWS, D), jnp.float32),
    compiler_params=pltpu.CompilerParams(needs_layout_passes=False),
    scratch_types=[
        pltpu.VMEM((_P, _B_W), jnp.int32),
        pltpu.VMEM((_B_W,), jnp.int32),
        pltpu.VMEM((_NCH, _CH, D), jnp.float32),
        pltpu.SemaphoreType.DMA,
        pltpu.SemaphoreType.DMA,
        pltpu.SemaphoreType.DMA,
        pltpu.SemaphoreType.DMA,
        pltpu.SemaphoreType.DMA,
        pltpu.SemaphoreType.DMA,
        pltpu.SemaphoreType.DMA,
    ],
)
def _sc_gather(pos_hbm, vn_hbm, out_hbm, pos_v, posm_v, rows_v,
               ps, gs0, gs1, gs2, gs3, ws0, ws1):
    c = lax.axis_index("c")
    s = lax.axis_index("s")
    wid = s * 2 + c
    base = wid * _B_W
    ph = [
        pltpu.async_copy(pos_hbm.at[p, pl.ds(base, _B_W)], pos_v.at[p], ps)
        for p in range(_P)
    ]
    for h in ph:
        h.wait()

    def merge_body(j, carry):
        dst = pl.multiple_of(j * _L, _L)
        m = pos_v[0, pl.ds(dst, _L)]
        for p in range(1, _P):
            m = jnp.maximum(m, pos_v[p, pl.ds(dst, _L)])
        posm_v[pl.ds(dst, _L)] = m
        return carry

    lax.fori_loop(0, _B_W // _L, merge_body, None)

    gsem = (gs0, gs1, gs2, gs3)
    wsem = (ws0, ws1)

    hg = [
        pltpu.async_copy(
            vn_hbm.at[posm_v.at[pl.ds(j * _CH, _CH)]], rows_v.at[j],
            gsem[j])
        for j in range(_NCH)
    ]
    hw = []
    for j in range(_NCH):
        hg[j].wait()
        hw.append(pltpu.async_copy(
            rows_v.at[j], out_hbm.at[pl.ds(base + j * _CH, _CH)],
            wsem[j & 1]))
    for h in hw:
        h.wait()


def kernel(mem, val, idx):
    del mem
    idx32 = idx.astype(jnp.int32)
    vn = _normalize(val)
    pos = _sc_pos(idx32)
    out = _sc_gather(pos, vn)
    return out
